# baseline (speedup 1.0000x reference)
import math

import jax
import jax.numpy as jnp
from jax import lax
from jax.experimental import pallas as pl
from jax.experimental.pallas import tpu as pltpu
from jax.experimental.pallas import tpu_sc as plsc

D_MODEL = 768
SCALE = math.sqrt(float(D_MODEL))
NUM_CORES = 2
NUM_SUBCORES = 16
NUM_WORKERS = NUM_CORES * NUM_SUBCORES
LANES = 16
CHUNK = 32
NBUF = 5
LA = 3


def _emb_body(idx_hbm, table_hbm, out_hbm, idx_v, rows_v, sem_g, sem_o):
    bsz, t = idx_hbm.shape
    n_idx = bsz * t
    rows_per_worker = n_idx // NUM_WORKERS
    n_chunks = rows_per_worker // CHUNK
    n_groups = n_chunks // NBUF
    w_per_row = t // rows_per_worker

    wid = lax.axis_index("s") * NUM_CORES + lax.axis_index("c")
    base = wid * rows_per_worker

    pltpu.sync_copy(
        idx_hbm.at[wid // w_per_row, pl.ds((wid % w_per_row) * rows_per_worker, rows_per_worker)],
        idx_v,
    )

    def issue_gather(c, b):
        pltpu.async_copy(
            table_hbm.at[idx_v.at[pl.ds(c * CHUNK, CHUNK)]],
            rows_v.at[b],
            sem_g.at[b],
        )

    def wait_gather(b):
        pltpu.make_async_copy(
            table_hbm.at[idx_v.at[pl.ds(0, CHUNK)]],
            rows_v.at[b],
            sem_g.at[b],
        ).wait()

    def issue_out(g, b):
        pltpu.async_copy(
            rows_v.at[b],
            out_hbm.at[pl.ds(base + g * CHUNK, CHUNK)],
            sem_o.at[b],
        )

    def wait_out(b):
        pltpu.make_async_copy(
            rows_v.at[b],
            out_hbm.at[pl.ds(base, CHUNK)],
            sem_o.at[b],
        ).wait()

    for c in range(LA):
        issue_gather(c, c % NBUF)

    @pl.loop(0, n_chunks)
    def chunk_body(g):
        b = lax.rem(g, NBUF)
        wait_gather(b)

        c = g + LA
        bc = lax.rem(c, NBUF)
        @pl.when(c < n_chunks)
        def _():
            @pl.when(c >= NBUF)
            def _():
                wait_out(bc)

            issue_gather(c, bc)

        half = CHUNK // 2

        @pl.loop(0, half)
        def scale_row_lo(r):
            @pl.loop(0, D_MODEL // LANES, unroll=8)
            def scale_slice(j):
                sl = pl.ds(j * LANES, LANES)
                rows_v[b, r, sl] = rows_v[b, r, sl] * SCALE

        pltpu.async_copy(
            rows_v.at[b, pl.ds(0, half)],
            out_hbm.at[pl.ds(base + g * CHUNK, half)],
            sem_o.at[b],
        )

        @pl.loop(half, CHUNK)
        def scale_row_hi(r):
            @pl.loop(0, D_MODEL // LANES, unroll=8)
            def scale_slice(j):
                sl = pl.ds(j * LANES, LANES)
                rows_v[b, r, sl] = rows_v[b, r, sl] * SCALE

        pltpu.async_copy(
            rows_v.at[b, pl.ds(half, half)],
            out_hbm.at[pl.ds(base + g * CHUNK + half, half)],
            sem_o.at[b],
        )

    for b in range(NBUF):
        wait_out(b)


@jax.jit
def kernel(input_ids, token_emb_weight):
    b, t = input_ids.shape
    n_idx = b * t
    ids32 = input_ids.astype(jnp.int32)

    grid_kernel = pl.kernel(
        _emb_body,
        out_type=jax.ShapeDtypeStruct((n_idx, D_MODEL), jnp.float32),
        mesh=plsc.VectorSubcoreMesh(
            core_axis_name="c",
            subcore_axis_name="s",
            num_cores=NUM_CORES,
            num_subcores=NUM_SUBCORES,
        ),
        scratch_types=[
            pltpu.VMEM((n_idx // NUM_WORKERS,), jnp.int32),
            pltpu.VMEM((NBUF, CHUNK, D_MODEL), jnp.float32),
            pltpu.SemaphoreType.DMA((NBUF,)),
            pltpu.SemaphoreType.DMA((NBUF,)),
        ],
    )
    out = grid_kernel(ids32, token_emb_weight)
    return out.reshape(b, t, D_MODEL)


# speedup vs baseline: 1.0167x; 1.0167x over previous
import math

import jax
import jax.numpy as jnp
from jax import lax
from jax.experimental import pallas as pl
from jax.experimental.pallas import tpu as pltpu
from jax.experimental.pallas import tpu_sc as plsc

D_MODEL = 768
SCALE = math.sqrt(float(D_MODEL))
NUM_CORES = 2
NUM_SUBCORES = 16
NUM_WORKERS = NUM_CORES * NUM_SUBCORES
LANES = 16
CHUNK = 32
NBUF = 4
LA = 3


def _emb_body(idx_hbm, table_hbm, out_hbm, idx_v, rows_v, sem_g, sem_o):
    bsz, t = idx_hbm.shape
    n_idx = bsz * t
    rows_per_worker = n_idx // NUM_WORKERS
    n_chunks = rows_per_worker // CHUNK
    n_groups = n_chunks // NBUF
    w_per_row = t // rows_per_worker

    wid = lax.axis_index("s") * NUM_CORES + lax.axis_index("c")
    base = wid * rows_per_worker

    pltpu.sync_copy(
        idx_hbm.at[wid // w_per_row, pl.ds((wid % w_per_row) * rows_per_worker, rows_per_worker)],
        idx_v,
    )

    def issue_gather(c, b):
        pltpu.async_copy(
            table_hbm.at[idx_v.at[pl.ds(c * CHUNK, CHUNK)]],
            rows_v.at[b],
            sem_g.at[b],
        )

    def wait_gather(b):
        pltpu.make_async_copy(
            table_hbm.at[idx_v.at[pl.ds(0, CHUNK)]],
            rows_v.at[b],
            sem_g.at[b],
        ).wait()

    def issue_out(g, b):
        pltpu.async_copy(
            rows_v.at[b],
            out_hbm.at[pl.ds(base + g * CHUNK, CHUNK)],
            sem_o.at[b],
        )

    def wait_out(b):
        pltpu.make_async_copy(
            rows_v.at[b],
            out_hbm.at[pl.ds(base, CHUNK)],
            sem_o.at[b],
        ).wait()

    for c in range(LA):
        issue_gather(c, c % NBUF)

    @pl.loop(0, n_chunks)
    def chunk_body(g):
        b = lax.rem(g, NBUF)
        wait_gather(b)

        c = g + LA
        bc = lax.rem(c, NBUF)
        @pl.when(c < n_chunks)
        def _():
            @pl.when(c >= NBUF)
            def _():
                wait_out(bc)

            issue_gather(c, bc)

        half = CHUNK // 2

        @pl.loop(0, half)
        def scale_row_lo(r):
            @pl.loop(0, D_MODEL // LANES, unroll=8)
            def scale_slice(j):
                sl = pl.ds(j * LANES, LANES)
                rows_v[b, r, sl] = rows_v[b, r, sl] * SCALE

        pltpu.async_copy(
            rows_v.at[b, pl.ds(0, half)],
            out_hbm.at[pl.ds(base + g * CHUNK, half)],
            sem_o.at[b],
        )

        @pl.loop(half, CHUNK)
        def scale_row_hi(r):
            @pl.loop(0, D_MODEL // LANES, unroll=8)
            def scale_slice(j):
                sl = pl.ds(j * LANES, LANES)
                rows_v[b, r, sl] = rows_v[b, r, sl] * SCALE

        pltpu.async_copy(
            rows_v.at[b, pl.ds(half, half)],
            out_hbm.at[pl.ds(base + g * CHUNK + half, half)],
            sem_o.at[b],
        )

    for b in range(NBUF):
        wait_out(b)


@jax.jit
def kernel(input_ids, token_emb_weight):
    b, t = input_ids.shape
    n_idx = b * t
    ids32 = input_ids.astype(jnp.int32)

    grid_kernel = pl.kernel(
        _emb_body,
        out_type=jax.ShapeDtypeStruct((n_idx, D_MODEL), jnp.float32),
        mesh=plsc.VectorSubcoreMesh(
            core_axis_name="c",
            subcore_axis_name="s",
            num_cores=NUM_CORES,
            num_subcores=NUM_SUBCORES,
        ),
        scratch_types=[
            pltpu.VMEM((n_idx // NUM_WORKERS,), jnp.int32),
            pltpu.VMEM((NBUF, CHUNK, D_MODEL), jnp.float32),
            pltpu.SemaphoreType.DMA((NBUF,)),
            pltpu.SemaphoreType.DMA((NBUF,)),
        ],
    )
    out = grid_kernel(ids32, token_emb_weight)
    return out.reshape(b, t, D_MODEL)
